# baseline (speedup 1.0000x reference)
import functools

import jax
import jax.numpy as jnp
from jax import lax
from jax.experimental import pallas as pl
from jax.experimental.pallas import tpu as pltpu
from jax.experimental.pallas import tpu_sc as plsc

_N = 10000
_E = 320000
_D = 128
_NC = 2
_NS = 16
_NW = _NC * _NS
_K = 80
_NCHUNK_TOT = _E // (_K * _NW)
_NBUF = 4
_NPT = 624
_NTAIL = _N - _NS * _NPT
_SLICES = ((0, 32), (32, 31), (63, 31), (94, 31))
_BE = 1280


@functools.cache
def _sc_mesh():
    return plsc.VectorSubcoreMesh(core_axis_name="c", subcore_axis_name="s",
                                  num_cores=_NC, num_subcores=_NS)


def _pre_body(h_ref, ws_ref, wt_ref, a_ref, b_ref):
    hb = h_ref[...]
    a_ref[...] = jnp.dot(hb, ws_ref[...], preferred_element_type=jnp.float32)
    b_ref[...] = jnp.dot(hb, wt_ref[...], preferred_element_type=jnp.float32)


_pre_call = pl.pallas_call(
    _pre_body,
    grid=(10,),
    in_specs=[
        pl.BlockSpec((_N // 10, _D), lambda i: (i, 0)),
        pl.BlockSpec((_D, _D), lambda i: (0, 0)),
        pl.BlockSpec((_D, _D), lambda i: (0, 0)),
    ],
    out_specs=[
        pl.BlockSpec((_N // 10, _D), lambda i: (i, 0)),
        pl.BlockSpec((_N // 10, _D), lambda i: (i, 0)),
    ],
    out_shape=[
        jax.ShapeDtypeStruct((_N, _D), jnp.float32),
        jax.ShapeDtypeStruct((_N, _D), jnp.float32),
    ],
)


@functools.cache
def _sc_gather_call(c0, nch):
    ne = nch * _K * _NW
    ngrp, nrem = nch // _NBUF, nch % _NBUF

    @functools.partial(
        pl.kernel,
        out_type=(
            jax.ShapeDtypeStruct((ne, _D), jnp.float32),
            jax.ShapeDtypeStruct((ne,), jnp.float32),
        ),
        mesh=_sc_mesh(),
        scratch_types=[
            pltpu.VMEM((_NBUF, _K), jnp.int32),
            pltpu.VMEM((_NBUF, _K), jnp.int32),
            pltpu.VMEM((_NBUF, _K, _D), jnp.float32),
            pltpu.VMEM((_NBUF, _K), jnp.float32),
            pltpu.VMEM((_N,), jnp.float32),
            pltpu.VMEM((_N,), jnp.float32),
            pltpu.VMEM((_N,), jnp.float32),
        ] + [pltpu.SemaphoreType.DMA] * (4 * _NBUF),
        compiler_params=pltpu.CompilerParams(needs_layout_passes=False),
    )
    def _sc_gather(a_hbm, b_hbm, cx_hbm, cy_hbm, cz_hbm, row_hbm, col_hbm,
                   s_hbm, rad_hbm,
                   idxr, idxc, bufs, radbuf, cxv, cyv, czv, *sems):
        isem = sems[0:_NBUF]
        gsemA = sems[_NBUF:2 * _NBUF]
        gsemB = sems[2 * _NBUF:3 * _NBUF]
        osem = sems[3 * _NBUF:4 * _NBUF]
        wid = lax.axis_index("s") * _NC + lax.axis_index("c")
        inbase = (c0 * _NW + wid * nch) * _K
        outbase = wid * nch * _K

        def idx_descs(c, b):
            off = inbase + c * _K
            return (pltpu.make_async_copy(row_hbm.at[pl.ds(off, _K)],
                                          idxr.at[b], isem[b]),
                    pltpu.make_async_copy(col_hbm.at[pl.ds(off, _K)],
                                          idxc.at[b], isem[b]))

        def a_desc(b):
            return pltpu.make_async_copy(a_hbm.at[idxr.at[b]], bufs.at[b],
                                         gsemA[b])

        def b_desc(b):
            return pltpu.make_async_copy(b_hbm.at[idxc.at[b]], bufs.at[b],
                                         gsemB[b])

        def out_descs(c, b):
            off = outbase + c * _K
            return (pltpu.make_async_copy(bufs.at[b], s_hbm.at[pl.ds(off, _K)],
                                          osem[b]),
                    pltpu.make_async_copy(radbuf.at[b],
                                          rad_hbm.at[pl.ds(off, _K)],
                                          osem[b]))


        def radial(b):
            for j in range(_K // 16):
                ir = idxr[b, pl.ds(j * 16, 16)]
                ic = idxc[b, pl.ds(j * 16, 16)]
                dx = plsc.load_gather(cxv, [ir]) - plsc.load_gather(cxv, [ic])
                dy = plsc.load_gather(cyv, [ir]) - plsc.load_gather(cyv, [ic])
                dz = plsc.load_gather(czv, [ir]) - plsc.load_gather(czv, [ic])
                radbuf[b, pl.ds(j * 16, 16)] = dx * dx + dy * dy + dz * dz

        pltpu.sync_copy(cx_hbm, cxv)
        pltpu.sync_copy(cy_hbm, cyv)
        pltpu.sync_copy(cz_hbm, czv)

        for d in idx_descs(0, 0):
            d.start()
            d.wait()
        for d in idx_descs(1, 1):
            d.start()
        a_desc(0).start()
        for d in idx_descs(1, 1):
            d.wait()
        a_desc(1).start()
        for d in idx_descs(2, 2):
            d.start()
        a_desc(0).wait()
        b_desc(0).start(add=True)

        def step(c, b):
            b_desc(b).wait()
            radial(b)
            for d in out_descs(c, b):
                d.start()

            @pl.when(c + 3 < nch)
            def _():
                for d in idx_descs(c + 3, (b + 3) % _NBUF):
                    d.start()

            b1 = (b + 1) % _NBUF
            b2 = (b + 2) % _NBUF

            @pl.when((c + 2 < nch) & (c >= 2))
            def _():
                for d in out_descs(c - 2, b2):
                    d.wait()

            @pl.when(c + 2 < nch)
            def _():
                for d in idx_descs(c + 2, b2):
                    d.wait()
                a_desc(b2).start()

            @pl.when(c + 1 < nch)
            def _():
                a_desc(b1).wait()
                b_desc(b1).start(add=True)

        def group(g, carry):
            for b in range(_NBUF):
                step(g * _NBUF + b, b)
            return carry

        lax.fori_loop(0, ngrp, group, 0)
        for r in range(nrem):
            c = ngrp * _NBUF + r
            step(c, c % _NBUF)
        for c in range(nch - _NBUF, nch):
            b = c % _NBUF
            for d in out_descs(c, b):
                d.wait()

    return _sc_gather


def _edge_body(s_ref, rad_ref, w2_ref, b1_ref, b2_ref, wr_ref,
               f_ref):
    radial = rad_ref[...]
    u = s_ref[...] + radial * wr_ref[...] + b1_ref[...]
    u = u * jax.nn.sigmoid(u)
    v = jnp.dot(u, w2_ref[...], preferred_element_type=jnp.float32) + b2_ref[...]
    f_ref[...] = v * jax.nn.sigmoid(v)


@functools.cache
def _edge_call(ne):
    return pl.pallas_call(
        _edge_body,
        grid=(ne // _BE,),
        in_specs=[
            pl.BlockSpec((_BE, _D), lambda i: (i, 0)),
            pl.BlockSpec((_BE, 1), lambda i: (i, 0)),
            pl.BlockSpec((_D, _D), lambda i: (0, 0)),
            pl.BlockSpec((1, _D), lambda i: (0, 0)),
            pl.BlockSpec((1, _D), lambda i: (0, 0)),
            pl.BlockSpec((1, _D), lambda i: (0, 0)),
        ],
        out_specs=pl.BlockSpec((_BE, _D), lambda i: (i, 0)),
        out_shape=jax.ShapeDtypeStruct((ne, _D), jnp.float32),
    )


@functools.cache
def _sc_scatter_call(c0, nch):
    ne = nch * _K * _NW
    ngrp, nrem = nch // _NBUF, nch % _NBUF

    @functools.partial(
        pl.kernel,
        out_type=jax.ShapeDtypeStruct((_NC * _N, _D), jnp.float32),
        mesh=_sc_mesh(),
        scratch_types=[
            pltpu.VMEM((_NBUF, _K), jnp.int32),
            pltpu.VMEM((_NBUF, _K, _D), jnp.float32),
            pltpu.VMEM_SHARED((_N, _D), jnp.float32),
        ] + [pltpu.SemaphoreType.DMA] * (2 * _NBUF),
    )
    def _sc_scatter(f_hbm, row_hbm, zero_hbm, agg_hbm, idx, buf, aggsh,
                    *sems):
        lsem = sems[0:_NBUF]
        ssem = sems[_NBUF:2 * _NBUF]
        c = lax.axis_index("c")
        s = lax.axis_index("s")
        wid = s * _NC + c
        inbase = (c0 * _NW + wid * nch) * _K
        fbase = wid * nch * _K

        def load_descs(ch, b):
            return (pltpu.make_async_copy(
                        row_hbm.at[pl.ds(inbase + ch * _K, _K)],
                        idx.at[b], lsem[b]),
                    pltpu.make_async_copy(
                        f_hbm.at[pl.ds(fbase + ch * _K, _K)],
                        buf.at[b], lsem[b]))

        def scat_desc(b):
            return pltpu.make_async_copy(buf.at[b], aggsh.at[idx.at[b]],
                                         ssem[b])

        pltpu.sync_copy(zero_hbm.at[pl.ds(s * _NPT, _NPT)],
                        aggsh.at[pl.ds(s * _NPT, _NPT)])

        @pl.when(s == _NS - 1)
        def _():
            pltpu.sync_copy(zero_hbm.at[pl.ds(_NS * _NPT, _NTAIL)],
                            aggsh.at[pl.ds(_NS * _NPT, _NTAIL)])

        plsc.subcore_barrier()

        for ch in (0, 1):
            for d in load_descs(ch, ch):
                d.start()

        def step(ch, b):
            for d in load_descs(ch, b):
                d.wait()
            scat_desc(b).start(add=True)
            b2 = (b + 2) % _NBUF

            @pl.when((ch + 2 < nch) & (ch >= 2))
            def _():
                scat_desc(b2).wait()

            @pl.when(ch + 2 < nch)
            def _():
                for d in load_descs(ch + 2, b2):
                    d.start()

        def group(g, carry):
            for b in range(_NBUF):
                step(g * _NBUF + b, b)
            return carry

        lax.fori_loop(0, ngrp, group, 0)
        for r in range(nrem):
            ch = ngrp * _NBUF + r
            step(ch, ch % _NBUF)
        for ch in range(nch - _NBUF, nch):
            scat_desc(ch % _NBUF).wait()
        plsc.subcore_barrier()
        pltpu.sync_copy(aggsh.at[pl.ds(s * _NPT, _NPT)],
                        agg_hbm.at[pl.ds(c * _N + s * _NPT, _NPT)])

        @pl.when(s == _NS - 1)
        def _():
            pltpu.sync_copy(aggsh.at[pl.ds(_NS * _NPT, _NTAIL)],
                            agg_hbm.at[pl.ds(c * _N + _NS * _NPT, _NTAIL)])

    return _sc_scatter


_NAGG = 2 * len(_SLICES)


def _node_body(*refs):
    h_ref = refs[0]
    agg_refs = refs[1:1 + _NAGG]
    w1h_ref, w1a_ref, b1_ref, w2_ref, b2_ref, o_ref = refs[1 + _NAGG:]
    hb = h_ref[...]
    agg = agg_refs[0][...]
    for a_ref in agg_refs[1:]:
        agg = agg + a_ref[...]
    u = (jnp.dot(hb, w1h_ref[...], preferred_element_type=jnp.float32)
         + jnp.dot(agg, w1a_ref[...], preferred_element_type=jnp.float32)
         + b1_ref[...])
    u = u * jax.nn.sigmoid(u)
    o_ref[...] = hb + jnp.dot(u, w2_ref[...],
                              preferred_element_type=jnp.float32) + b2_ref[...]


_node_call = pl.pallas_call(
    _node_body,
    grid=(10,),
    in_specs=[pl.BlockSpec((_N // 10, _D), lambda i: (i, 0))] * (1 + _NAGG) + [
        pl.BlockSpec((_D, _D), lambda i: (0, 0)),
        pl.BlockSpec((_D, _D), lambda i: (0, 0)),
        pl.BlockSpec((1, _D), lambda i: (0, 0)),
        pl.BlockSpec((_D, _D), lambda i: (0, 0)),
        pl.BlockSpec((1, _D), lambda i: (0, 0)),
    ],
    out_specs=pl.BlockSpec((_N // 10, _D), lambda i: (i, 0)),
    out_shape=jax.ShapeDtypeStruct((_N, _D), jnp.float32),
)


def kernel(h, edge_index, coord, W_e1, b_e1, W_e2, b_e2, W_n1, b_n1, W_n2,
           b_n2):
    row = edge_index[0]
    col = edge_index[1]
    zeros = jnp.zeros((_N, _D), jnp.float32)
    b1 = b_e1.reshape(1, _D)
    b2 = b_e2.reshape(1, _D)
    wr = W_e1[2 * _D:2 * _D + 1]

    A, B = _pre_call(h, W_e1[0:_D], W_e1[_D:2 * _D])
    aggs = []
    for c0, nch in _SLICES:
        ne = nch * _K * _NW
        ST, rad = _sc_gather_call(c0, nch)(
            A, B, coord[:, 0], coord[:, 1], coord[:, 2], row, col)
        f = _edge_call(ne)(ST, rad.reshape(ne, 1), W_e2, b1, b2, wr)
        agg2 = _sc_scatter_call(c0, nch)(f, row, zeros)
        aggs += [agg2[:_N], agg2[_N:]]
    out = _node_call(h, *aggs, W_n1[:_D], W_n1[_D:], b_n1.reshape(1, _D),
                     W_n2, b_n2.reshape(1, _D))
    return out


# speedup vs baseline: 1.0121x; 1.0121x over previous
import functools

import jax
import jax.numpy as jnp
from jax import lax
from jax.experimental import pallas as pl
from jax.experimental.pallas import tpu as pltpu
from jax.experimental.pallas import tpu_sc as plsc

_N = 10000
_E = 320000
_D = 128
_NC = 2
_NS = 16
_NW = _NC * _NS
_K = 80
_NCHUNK_TOT = _E // (_K * _NW)
_NBUF = 4
_NPT = 624
_NTAIL = _N - _NS * _NPT
_SLICES = ((0, 42), (42, 42), (84, 41))
_BE = 1280


@functools.cache
def _sc_mesh():
    return plsc.VectorSubcoreMesh(core_axis_name="c", subcore_axis_name="s",
                                  num_cores=_NC, num_subcores=_NS)


def _pre_body(h_ref, ws_ref, wt_ref, a_ref, b_ref):
    hb = h_ref[...]
    a_ref[...] = jnp.dot(hb, ws_ref[...], preferred_element_type=jnp.float32)
    b_ref[...] = jnp.dot(hb, wt_ref[...], preferred_element_type=jnp.float32)


_pre_call = pl.pallas_call(
    _pre_body,
    grid=(10,),
    in_specs=[
        pl.BlockSpec((_N // 10, _D), lambda i: (i, 0)),
        pl.BlockSpec((_D, _D), lambda i: (0, 0)),
        pl.BlockSpec((_D, _D), lambda i: (0, 0)),
    ],
    out_specs=[
        pl.BlockSpec((_N // 10, _D), lambda i: (i, 0)),
        pl.BlockSpec((_N // 10, _D), lambda i: (i, 0)),
    ],
    out_shape=[
        jax.ShapeDtypeStruct((_N, _D), jnp.float32),
        jax.ShapeDtypeStruct((_N, _D), jnp.float32),
    ],
)


@functools.cache
def _sc_gather_call(c0, nch):
    ne = nch * _K * _NW
    ngrp, nrem = nch // _NBUF, nch % _NBUF

    @functools.partial(
        pl.kernel,
        out_type=(
            jax.ShapeDtypeStruct((ne, _D), jnp.float32),
            jax.ShapeDtypeStruct((ne,), jnp.float32),
        ),
        mesh=_sc_mesh(),
        scratch_types=[
            pltpu.VMEM((_NBUF, _K), jnp.int32),
            pltpu.VMEM((_NBUF, _K), jnp.int32),
            pltpu.VMEM((_NBUF, _K, _D), jnp.float32),
            pltpu.VMEM((_NBUF, _K), jnp.float32),
            pltpu.VMEM((_N,), jnp.float32),
            pltpu.VMEM((_N,), jnp.float32),
            pltpu.VMEM((_N,), jnp.float32),
        ] + [pltpu.SemaphoreType.DMA] * (4 * _NBUF),
        compiler_params=pltpu.CompilerParams(needs_layout_passes=False),
    )
    def _sc_gather(a_hbm, b_hbm, cx_hbm, cy_hbm, cz_hbm, row_hbm, col_hbm,
                   s_hbm, rad_hbm,
                   idxr, idxc, bufs, radbuf, cxv, cyv, czv, *sems):
        isem = sems[0:_NBUF]
        gsemA = sems[_NBUF:2 * _NBUF]
        gsemB = sems[2 * _NBUF:3 * _NBUF]
        osem = sems[3 * _NBUF:4 * _NBUF]
        wid = lax.axis_index("s") * _NC + lax.axis_index("c")
        inbase = (c0 * _NW + wid * nch) * _K
        outbase = wid * nch * _K

        def idx_descs(c, b):
            off = inbase + c * _K
            return (pltpu.make_async_copy(row_hbm.at[pl.ds(off, _K)],
                                          idxr.at[b], isem[b]),
                    pltpu.make_async_copy(col_hbm.at[pl.ds(off, _K)],
                                          idxc.at[b], isem[b]))

        def a_desc(b):
            return pltpu.make_async_copy(a_hbm.at[idxr.at[b]], bufs.at[b],
                                         gsemA[b])

        def b_desc(b):
            return pltpu.make_async_copy(b_hbm.at[idxc.at[b]], bufs.at[b],
                                         gsemB[b])

        def out_descs(c, b):
            off = outbase + c * _K
            return (pltpu.make_async_copy(bufs.at[b], s_hbm.at[pl.ds(off, _K)],
                                          osem[b]),
                    pltpu.make_async_copy(radbuf.at[b],
                                          rad_hbm.at[pl.ds(off, _K)],
                                          osem[b]))


        def radial(b):
            for j in range(_K // 16):
                ir = idxr[b, pl.ds(j * 16, 16)]
                ic = idxc[b, pl.ds(j * 16, 16)]
                dx = plsc.load_gather(cxv, [ir]) - plsc.load_gather(cxv, [ic])
                dy = plsc.load_gather(cyv, [ir]) - plsc.load_gather(cyv, [ic])
                dz = plsc.load_gather(czv, [ir]) - plsc.load_gather(czv, [ic])
                radbuf[b, pl.ds(j * 16, 16)] = dx * dx + dy * dy + dz * dz

        pltpu.sync_copy(cx_hbm, cxv)
        pltpu.sync_copy(cy_hbm, cyv)
        pltpu.sync_copy(cz_hbm, czv)

        for d in idx_descs(0, 0):
            d.start()
            d.wait()
        for d in idx_descs(1, 1):
            d.start()
        a_desc(0).start()
        for d in idx_descs(1, 1):
            d.wait()
        a_desc(1).start()
        for d in idx_descs(2, 2):
            d.start()
        a_desc(0).wait()
        b_desc(0).start(add=True)

        def step(c, b):
            b_desc(b).wait()
            radial(b)
            for d in out_descs(c, b):
                d.start()

            @pl.when(c + 3 < nch)
            def _():
                for d in idx_descs(c + 3, (b + 3) % _NBUF):
                    d.start()

            b1 = (b + 1) % _NBUF
            b2 = (b + 2) % _NBUF

            @pl.when((c + 2 < nch) & (c >= 2))
            def _():
                for d in out_descs(c - 2, b2):
                    d.wait()

            @pl.when(c + 2 < nch)
            def _():
                for d in idx_descs(c + 2, b2):
                    d.wait()
                a_desc(b2).start()

            @pl.when(c + 1 < nch)
            def _():
                a_desc(b1).wait()
                b_desc(b1).start(add=True)

        def group(g, carry):
            for b in range(_NBUF):
                step(g * _NBUF + b, b)
            return carry

        lax.fori_loop(0, ngrp, group, 0)
        for r in range(nrem):
            c = ngrp * _NBUF + r
            step(c, c % _NBUF)
        for c in range(nch - _NBUF, nch):
            b = c % _NBUF
            for d in out_descs(c, b):
                d.wait()

    return _sc_gather


def _edge_body(s_ref, rad_ref, w2_ref, b1_ref, b2_ref, wr_ref,
               f_ref):
    radial = rad_ref[...]
    u = s_ref[...] + radial * wr_ref[...] + b1_ref[...]
    u = u * jax.nn.sigmoid(u)
    v = jnp.dot(u, w2_ref[...], preferred_element_type=jnp.float32) + b2_ref[...]
    f_ref[...] = v * jax.nn.sigmoid(v)


@functools.cache
def _edge_call(ne):
    return pl.pallas_call(
        _edge_body,
        grid=(ne // _BE,),
        in_specs=[
            pl.BlockSpec((_BE, _D), lambda i: (i, 0)),
            pl.BlockSpec((_BE, 1), lambda i: (i, 0)),
            pl.BlockSpec((_D, _D), lambda i: (0, 0)),
            pl.BlockSpec((1, _D), lambda i: (0, 0)),
            pl.BlockSpec((1, _D), lambda i: (0, 0)),
            pl.BlockSpec((1, _D), lambda i: (0, 0)),
        ],
        out_specs=pl.BlockSpec((_BE, _D), lambda i: (i, 0)),
        out_shape=jax.ShapeDtypeStruct((ne, _D), jnp.float32),
    )


@functools.cache
def _sc_scatter_call(c0, nch):
    ne = nch * _K * _NW
    ngrp, nrem = nch // _NBUF, nch % _NBUF

    @functools.partial(
        pl.kernel,
        out_type=jax.ShapeDtypeStruct((_NC * _N, _D), jnp.float32),
        mesh=_sc_mesh(),
        scratch_types=[
            pltpu.VMEM((_NBUF, _K), jnp.int32),
            pltpu.VMEM((_NBUF, _K, _D), jnp.float32),
            pltpu.VMEM_SHARED((_N, _D), jnp.float32),
        ] + [pltpu.SemaphoreType.DMA] * (2 * _NBUF),
    )
    def _sc_scatter(f_hbm, row_hbm, zero_hbm, agg_hbm, idx, buf, aggsh,
                    *sems):
        lsem = sems[0:_NBUF]
        ssem = sems[_NBUF:2 * _NBUF]
        c = lax.axis_index("c")
        s = lax.axis_index("s")
        wid = s * _NC + c
        inbase = (c0 * _NW + wid * nch) * _K
        fbase = wid * nch * _K

        def load_descs(ch, b):
            return (pltpu.make_async_copy(
                        row_hbm.at[pl.ds(inbase + ch * _K, _K)],
                        idx.at[b], lsem[b]),
                    pltpu.make_async_copy(
                        f_hbm.at[pl.ds(fbase + ch * _K, _K)],
                        buf.at[b], lsem[b]))

        def scat_desc(b):
            return pltpu.make_async_copy(buf.at[b], aggsh.at[idx.at[b]],
                                         ssem[b])

        pltpu.sync_copy(zero_hbm.at[pl.ds(s * _NPT, _NPT)],
                        aggsh.at[pl.ds(s * _NPT, _NPT)])

        @pl.when(s == _NS - 1)
        def _():
            pltpu.sync_copy(zero_hbm.at[pl.ds(_NS * _NPT, _NTAIL)],
                            aggsh.at[pl.ds(_NS * _NPT, _NTAIL)])

        plsc.subcore_barrier()

        for ch in (0, 1):
            for d in load_descs(ch, ch):
                d.start()

        def step(ch, b):
            for d in load_descs(ch, b):
                d.wait()
            scat_desc(b).start(add=True)
            b2 = (b + 2) % _NBUF

            @pl.when((ch + 2 < nch) & (ch >= 2))
            def _():
                scat_desc(b2).wait()

            @pl.when(ch + 2 < nch)
            def _():
                for d in load_descs(ch + 2, b2):
                    d.start()

        def group(g, carry):
            for b in range(_NBUF):
                step(g * _NBUF + b, b)
            return carry

        lax.fori_loop(0, ngrp, group, 0)
        for r in range(nrem):
            ch = ngrp * _NBUF + r
            step(ch, ch % _NBUF)
        for ch in range(nch - _NBUF, nch):
            scat_desc(ch % _NBUF).wait()
        plsc.subcore_barrier()
        pltpu.sync_copy(aggsh.at[pl.ds(s * _NPT, _NPT)],
                        agg_hbm.at[pl.ds(c * _N + s * _NPT, _NPT)])

        @pl.when(s == _NS - 1)
        def _():
            pltpu.sync_copy(aggsh.at[pl.ds(_NS * _NPT, _NTAIL)],
                            agg_hbm.at[pl.ds(c * _N + _NS * _NPT, _NTAIL)])

    return _sc_scatter


_NAGG = 2 * len(_SLICES)


def _node_body(*refs):
    h_ref = refs[0]
    agg_refs = refs[1:1 + _NAGG]
    w1h_ref, w1a_ref, b1_ref, w2_ref, b2_ref, o_ref = refs[1 + _NAGG:]
    hb = h_ref[...]
    agg = agg_refs[0][...]
    for a_ref in agg_refs[1:]:
        agg = agg + a_ref[...]
    u = (jnp.dot(hb, w1h_ref[...], preferred_element_type=jnp.float32)
         + jnp.dot(agg, w1a_ref[...], preferred_element_type=jnp.float32)
         + b1_ref[...])
    u = u * jax.nn.sigmoid(u)
    o_ref[...] = hb + jnp.dot(u, w2_ref[...],
                              preferred_element_type=jnp.float32) + b2_ref[...]


_node_call = pl.pallas_call(
    _node_body,
    grid=(10,),
    in_specs=[pl.BlockSpec((_N // 10, _D), lambda i: (i, 0))] * (1 + _NAGG) + [
        pl.BlockSpec((_D, _D), lambda i: (0, 0)),
        pl.BlockSpec((_D, _D), lambda i: (0, 0)),
        pl.BlockSpec((1, _D), lambda i: (0, 0)),
        pl.BlockSpec((_D, _D), lambda i: (0, 0)),
        pl.BlockSpec((1, _D), lambda i: (0, 0)),
    ],
    out_specs=pl.BlockSpec((_N // 10, _D), lambda i: (i, 0)),
    out_shape=jax.ShapeDtypeStruct((_N, _D), jnp.float32),
)


def kernel(h, edge_index, coord, W_e1, b_e1, W_e2, b_e2, W_n1, b_n1, W_n2,
           b_n2):
    row = edge_index[0]
    col = edge_index[1]
    zeros = jnp.zeros((_N, _D), jnp.float32)
    b1 = b_e1.reshape(1, _D)
    b2 = b_e2.reshape(1, _D)
    wr = W_e1[2 * _D:2 * _D + 1]

    A, B = _pre_call(h, W_e1[0:_D], W_e1[_D:2 * _D])
    aggs = []
    for c0, nch in _SLICES:
        ne = nch * _K * _NW
        ST, rad = _sc_gather_call(c0, nch)(
            A, B, coord[:, 0], coord[:, 1], coord[:, 2], row, col)
        f = _edge_call(ne)(ST, rad.reshape(ne, 1), W_e2, b1, b2, wr)
        agg2 = _sc_scatter_call(c0, nch)(f, row, zeros)
        aggs += [agg2[:_N], agg2[_N:]]
    out = _node_call(h, *aggs, W_n1[:_D], W_n1[_D:], b_n1.reshape(1, _D),
                     W_n2, b_n2.reshape(1, _D))
    return out


# speedup vs baseline: 1.1023x; 1.0892x over previous
import functools

import jax
import jax.numpy as jnp
from jax import lax
from jax.experimental import pallas as pl
from jax.experimental.pallas import tpu as pltpu
from jax.experimental.pallas import tpu_sc as plsc

_N = 10000
_E = 320000
_D = 128
_NC = 2
_NS = 16
_NW = _NC * _NS
_K = 80
_NCHUNK_TOT = _E // (_K * _NW)
_NBUF = 4
_NPT = 624
_NTAIL = _N - _NS * _NPT
_SLICES = ((0, 42), (42, 42), (84, 41))
_BE = 2560


@functools.cache
def _sc_mesh():
    return plsc.VectorSubcoreMesh(core_axis_name="c", subcore_axis_name="s",
                                  num_cores=_NC, num_subcores=_NS)


def _pre_body(h_ref, ws_ref, wt_ref, a_ref, b_ref):
    hb = h_ref[...]
    a_ref[...] = jnp.dot(hb, ws_ref[...], preferred_element_type=jnp.float32)
    b_ref[...] = jnp.dot(hb, wt_ref[...], preferred_element_type=jnp.float32)


_pre_call = pl.pallas_call(
    _pre_body,
    grid=(10,),
    in_specs=[
        pl.BlockSpec((_N // 10, _D), lambda i: (i, 0)),
        pl.BlockSpec((_D, _D), lambda i: (0, 0)),
        pl.BlockSpec((_D, _D), lambda i: (0, 0)),
    ],
    out_specs=[
        pl.BlockSpec((_N // 10, _D), lambda i: (i, 0)),
        pl.BlockSpec((_N // 10, _D), lambda i: (i, 0)),
    ],
    out_shape=[
        jax.ShapeDtypeStruct((_N, _D), jnp.float32),
        jax.ShapeDtypeStruct((_N, _D), jnp.float32),
    ],
)


@functools.cache
def _sc_gather_call(c0, nch):
    ne = nch * _K * _NW
    ngrp, nrem = nch // _NBUF, nch % _NBUF

    @functools.partial(
        pl.kernel,
        out_type=(
            jax.ShapeDtypeStruct((ne, _D), jnp.float32),
            jax.ShapeDtypeStruct((ne,), jnp.float32),
        ),
        mesh=_sc_mesh(),
        scratch_types=[
            pltpu.VMEM((_NBUF, _K), jnp.int32),
            pltpu.VMEM((_NBUF, _K), jnp.int32),
            pltpu.VMEM((_NBUF, _K, _D), jnp.float32),
            pltpu.VMEM((_NBUF, _K), jnp.float32),
            pltpu.VMEM((_N,), jnp.float32),
            pltpu.VMEM((_N,), jnp.float32),
            pltpu.VMEM((_N,), jnp.float32),
        ] + [pltpu.SemaphoreType.DMA] * (4 * _NBUF),
        compiler_params=pltpu.CompilerParams(needs_layout_passes=False),
    )
    def _sc_gather(a_hbm, b_hbm, cx_hbm, cy_hbm, cz_hbm, row_hbm, col_hbm,
                   s_hbm, rad_hbm,
                   idxr, idxc, bufs, radbuf, cxv, cyv, czv, *sems):
        isem = sems[0:_NBUF]
        gsemA = sems[_NBUF:2 * _NBUF]
        gsemB = sems[2 * _NBUF:3 * _NBUF]
        osem = sems[3 * _NBUF:4 * _NBUF]
        wid = lax.axis_index("s") * _NC + lax.axis_index("c")
        inbase = (c0 * _NW + wid * nch) * _K
        outbase = wid * nch * _K

        def idx_descs(c, b):
            off = inbase + c * _K
            return (pltpu.make_async_copy(row_hbm.at[pl.ds(off, _K)],
                                          idxr.at[b], isem[b]),
                    pltpu.make_async_copy(col_hbm.at[pl.ds(off, _K)],
                                          idxc.at[b], isem[b]))

        def a_desc(b):
            return pltpu.make_async_copy(a_hbm.at[idxr.at[b]], bufs.at[b],
                                         gsemA[b])

        def b_desc(b):
            return pltpu.make_async_copy(b_hbm.at[idxc.at[b]], bufs.at[b],
                                         gsemB[b])

        def out_descs(c, b):
            off = outbase + c * _K
            return (pltpu.make_async_copy(bufs.at[b], s_hbm.at[pl.ds(off, _K)],
                                          osem[b]),
                    pltpu.make_async_copy(radbuf.at[b],
                                          rad_hbm.at[pl.ds(off, _K)],
                                          osem[b]))


        def radial(b):
            for j in range(_K // 16):
                ir = idxr[b, pl.ds(j * 16, 16)]
                ic = idxc[b, pl.ds(j * 16, 16)]
                dx = plsc.load_gather(cxv, [ir]) - plsc.load_gather(cxv, [ic])
                dy = plsc.load_gather(cyv, [ir]) - plsc.load_gather(cyv, [ic])
                dz = plsc.load_gather(czv, [ir]) - plsc.load_gather(czv, [ic])
                radbuf[b, pl.ds(j * 16, 16)] = dx * dx + dy * dy + dz * dz

        pltpu.sync_copy(cx_hbm, cxv)
        pltpu.sync_copy(cy_hbm, cyv)
        pltpu.sync_copy(cz_hbm, czv)

        for d in idx_descs(0, 0):
            d.start()
            d.wait()
        for d in idx_descs(1, 1):
            d.start()
        a_desc(0).start()
        for d in idx_descs(1, 1):
            d.wait()
        a_desc(1).start()
        for d in idx_descs(2, 2):
            d.start()
        a_desc(0).wait()
        b_desc(0).start(add=True)

        def step(c, b):
            b_desc(b).wait()
            radial(b)
            for d in out_descs(c, b):
                d.start()

            @pl.when(c + 3 < nch)
            def _():
                for d in idx_descs(c + 3, (b + 3) % _NBUF):
                    d.start()

            b1 = (b + 1) % _NBUF
            b2 = (b + 2) % _NBUF

            @pl.when((c + 2 < nch) & (c >= 2))
            def _():
                for d in out_descs(c - 2, b2):
                    d.wait()

            @pl.when(c + 2 < nch)
            def _():
                for d in idx_descs(c + 2, b2):
                    d.wait()
                a_desc(b2).start()

            @pl.when(c + 1 < nch)
            def _():
                a_desc(b1).wait()
                b_desc(b1).start(add=True)

        def group(g, carry):
            for b in range(_NBUF):
                step(g * _NBUF + b, b)
            return carry

        lax.fori_loop(0, ngrp, group, 0)
        for r in range(nrem):
            c = ngrp * _NBUF + r
            step(c, c % _NBUF)
        for c in range(nch - _NBUF, nch):
            b = c % _NBUF
            for d in out_descs(c, b):
                d.wait()

    return _sc_gather


def _edge_body(s_ref, rad_ref, w2_ref, b1_ref, b2_ref, wr_ref,
               f_ref):
    radial = rad_ref[...]
    u = s_ref[...] + radial * wr_ref[...] + b1_ref[...]
    u = u * jax.nn.sigmoid(u)
    v = jnp.dot(u, w2_ref[...], preferred_element_type=jnp.float32) + b2_ref[...]
    f_ref[...] = v * jax.nn.sigmoid(v)


@functools.cache
def _edge_call(ne):
    return pl.pallas_call(
        _edge_body,
        grid=(ne // _BE,),
        in_specs=[
            pl.BlockSpec((_BE, _D), lambda i: (i, 0)),
            pl.BlockSpec((_BE, 1), lambda i: (i, 0)),
            pl.BlockSpec((_D, _D), lambda i: (0, 0)),
            pl.BlockSpec((1, _D), lambda i: (0, 0)),
            pl.BlockSpec((1, _D), lambda i: (0, 0)),
            pl.BlockSpec((1, _D), lambda i: (0, 0)),
        ],
        out_specs=pl.BlockSpec((_BE, _D), lambda i: (i, 0)),
        out_shape=jax.ShapeDtypeStruct((ne, _D), jnp.float32),
    )


@functools.cache
def _sc_scatter_call(c0, nch):
    ne = nch * _K * _NW
    ngrp, nrem = nch // _NBUF, nch % _NBUF

    @functools.partial(
        pl.kernel,
        out_type=jax.ShapeDtypeStruct((_NC * _N, _D), jnp.float32),
        mesh=_sc_mesh(),
        scratch_types=[
            pltpu.VMEM((_NBUF, _K), jnp.int32),
            pltpu.VMEM((_NBUF, _K, _D), jnp.float32),
            pltpu.VMEM_SHARED((_N, _D), jnp.float32),
        ] + [pltpu.SemaphoreType.DMA] * (2 * _NBUF),
    )
    def _sc_scatter(f_hbm, row_hbm, zero_hbm, agg_hbm, idx, buf, aggsh,
                    *sems):
        lsem = sems[0:_NBUF]
        ssem = sems[_NBUF:2 * _NBUF]
        c = lax.axis_index("c")
        s = lax.axis_index("s")
        wid = s * _NC + c
        inbase = (c0 * _NW + wid * nch) * _K
        fbase = wid * nch * _K

        def load_descs(ch, b):
            return (pltpu.make_async_copy(
                        row_hbm.at[pl.ds(inbase + ch * _K, _K)],
                        idx.at[b], lsem[b]),
                    pltpu.make_async_copy(
                        f_hbm.at[pl.ds(fbase + ch * _K, _K)],
                        buf.at[b], lsem[b]))

        def scat_desc(b):
            return pltpu.make_async_copy(buf.at[b], aggsh.at[idx.at[b]],
                                         ssem[b])

        pltpu.sync_copy(zero_hbm.at[pl.ds(s * _NPT, _NPT)],
                        aggsh.at[pl.ds(s * _NPT, _NPT)])

        @pl.when(s == _NS - 1)
        def _():
            pltpu.sync_copy(zero_hbm.at[pl.ds(_NS * _NPT, _NTAIL)],
                            aggsh.at[pl.ds(_NS * _NPT, _NTAIL)])

        plsc.subcore_barrier()

        for ch in (0, 1):
            for d in load_descs(ch, ch):
                d.start()

        def step(ch, b):
            for d in load_descs(ch, b):
                d.wait()
            scat_desc(b).start(add=True)
            b2 = (b + 2) % _NBUF

            @pl.when((ch + 2 < nch) & (ch >= 2))
            def _():
                scat_desc(b2).wait()

            @pl.when(ch + 2 < nch)
            def _():
                for d in load_descs(ch + 2, b2):
                    d.start()

        def group(g, carry):
            for b in range(_NBUF):
                step(g * _NBUF + b, b)
            return carry

        lax.fori_loop(0, ngrp, group, 0)
        for r in range(nrem):
            ch = ngrp * _NBUF + r
            step(ch, ch % _NBUF)
        for ch in range(nch - _NBUF, nch):
            scat_desc(ch % _NBUF).wait()
        plsc.subcore_barrier()
        pltpu.sync_copy(aggsh.at[pl.ds(s * _NPT, _NPT)],
                        agg_hbm.at[pl.ds(c * _N + s * _NPT, _NPT)])

        @pl.when(s == _NS - 1)
        def _():
            pltpu.sync_copy(aggsh.at[pl.ds(_NS * _NPT, _NTAIL)],
                            agg_hbm.at[pl.ds(c * _N + _NS * _NPT, _NTAIL)])

    return _sc_scatter


_NAGG = 2 * len(_SLICES)


def _node_body(*refs):
    h_ref = refs[0]
    agg_refs = refs[1:1 + _NAGG]
    w1h_ref, w1a_ref, b1_ref, w2_ref, b2_ref, o_ref = refs[1 + _NAGG:]
    hb = h_ref[...]
    agg = agg_refs[0][...]
    for a_ref in agg_refs[1:]:
        agg = agg + a_ref[...]
    u = (jnp.dot(hb, w1h_ref[...], preferred_element_type=jnp.float32)
         + jnp.dot(agg, w1a_ref[...], preferred_element_type=jnp.float32)
         + b1_ref[...])
    u = u * jax.nn.sigmoid(u)
    o_ref[...] = hb + jnp.dot(u, w2_ref[...],
                              preferred_element_type=jnp.float32) + b2_ref[...]


_node_call = pl.pallas_call(
    _node_body,
    grid=(10,),
    in_specs=[pl.BlockSpec((_N // 10, _D), lambda i: (i, 0))] * (1 + _NAGG) + [
        pl.BlockSpec((_D, _D), lambda i: (0, 0)),
        pl.BlockSpec((_D, _D), lambda i: (0, 0)),
        pl.BlockSpec((1, _D), lambda i: (0, 0)),
        pl.BlockSpec((_D, _D), lambda i: (0, 0)),
        pl.BlockSpec((1, _D), lambda i: (0, 0)),
    ],
    out_specs=pl.BlockSpec((_N // 10, _D), lambda i: (i, 0)),
    out_shape=jax.ShapeDtypeStruct((_N, _D), jnp.float32),
)


def kernel(h, edge_index, coord, W_e1, b_e1, W_e2, b_e2, W_n1, b_n1, W_n2,
           b_n2):
    row = edge_index[0]
    col = edge_index[1]
    zeros = jnp.zeros((_N, _D), jnp.float32)
    b1 = b_e1.reshape(1, _D)
    b2 = b_e2.reshape(1, _D)
    wr = W_e1[2 * _D:2 * _D + 1]

    A, B = _pre_call(h, W_e1[0:_D], W_e1[_D:2 * _D])
    aggs = []
    for c0, nch in _SLICES:
        ne = nch * _K * _NW
        ST, rad = _sc_gather_call(c0, nch)(
            A, B, coord[:, 0], coord[:, 1], coord[:, 2], row, col)
        f = _edge_call(ne)(ST, rad.reshape(ne, 1), W_e2, b1, b2, wr)
        agg2 = _sc_scatter_call(c0, nch)(f, row, zeros)
        aggs += [agg2[:_N], agg2[_N:]]
    out = _node_call(h, *aggs, W_n1[:_D], W_n1[_D:], b_n1.reshape(1, _D),
                     W_n2, b_n2.reshape(1, _D))
    return out


# speedup vs baseline: 1.1116x; 1.0085x over previous
import functools

import jax
import jax.numpy as jnp
from jax import lax
from jax.experimental import pallas as pl
from jax.experimental.pallas import tpu as pltpu
from jax.experimental.pallas import tpu_sc as plsc

_N = 10000
_E = 320000
_D = 128
_NC = 2
_NS = 16
_NW = _NC * _NS
_K = 80
_NCHUNK_TOT = _E // (_K * _NW)
_NBUF = 4
_NPT = 624
_NTAIL = _N - _NS * _NPT
_SLICES = ((0, 42), (42, 42), (84, 41))
_BE = 2560


@functools.cache
def _sc_mesh():
    return plsc.VectorSubcoreMesh(core_axis_name="c", subcore_axis_name="s",
                                  num_cores=_NC, num_subcores=_NS)


def _pre_body(h_ref, ws_ref, wt_ref, a_ref, b_ref):
    hb = h_ref[...]
    a_ref[...] = jnp.dot(hb, ws_ref[...], preferred_element_type=jnp.float32)
    b_ref[...] = jnp.dot(hb, wt_ref[...], preferred_element_type=jnp.float32)


_pre_call = pl.pallas_call(
    _pre_body,
    grid=(5,),
    in_specs=[
        pl.BlockSpec((_N // 5, _D), lambda i: (i, 0)),
        pl.BlockSpec((_D, _D), lambda i: (0, 0)),
        pl.BlockSpec((_D, _D), lambda i: (0, 0)),
    ],
    out_specs=[
        pl.BlockSpec((_N // 5, _D), lambda i: (i, 0)),
        pl.BlockSpec((_N // 5, _D), lambda i: (i, 0)),
    ],
    out_shape=[
        jax.ShapeDtypeStruct((_N, _D), jnp.float32),
        jax.ShapeDtypeStruct((_N, _D), jnp.float32),
    ],
)


@functools.cache
def _sc_gather_call(c0, nch):
    ne = nch * _K * _NW
    ngrp, nrem = nch // _NBUF, nch % _NBUF

    @functools.partial(
        pl.kernel,
        out_type=(
            jax.ShapeDtypeStruct((ne, _D), jnp.float32),
            jax.ShapeDtypeStruct((ne,), jnp.float32),
        ),
        mesh=_sc_mesh(),
        scratch_types=[
            pltpu.VMEM((_NBUF, _K), jnp.int32),
            pltpu.VMEM((_NBUF, _K), jnp.int32),
            pltpu.VMEM((_NBUF, _K, _D), jnp.float32),
            pltpu.VMEM((_NBUF, _K), jnp.float32),
            pltpu.VMEM((_N,), jnp.float32),
            pltpu.VMEM((_N,), jnp.float32),
            pltpu.VMEM((_N,), jnp.float32),
        ] + [pltpu.SemaphoreType.DMA] * (4 * _NBUF),
        compiler_params=pltpu.CompilerParams(needs_layout_passes=False),
    )
    def _sc_gather(a_hbm, b_hbm, cx_hbm, cy_hbm, cz_hbm, row_hbm, col_hbm,
                   s_hbm, rad_hbm,
                   idxr, idxc, bufs, radbuf, cxv, cyv, czv, *sems):
        isem = sems[0:_NBUF]
        gsemA = sems[_NBUF:2 * _NBUF]
        gsemB = sems[2 * _NBUF:3 * _NBUF]
        osem = sems[3 * _NBUF:4 * _NBUF]
        wid = lax.axis_index("s") * _NC + lax.axis_index("c")
        inbase = (c0 * _NW + wid * nch) * _K
        outbase = wid * nch * _K

        def idx_descs(c, b):
            off = inbase + c * _K
            return (pltpu.make_async_copy(row_hbm.at[pl.ds(off, _K)],
                                          idxr.at[b], isem[b]),
                    pltpu.make_async_copy(col_hbm.at[pl.ds(off, _K)],
                                          idxc.at[b], isem[b]))

        def a_desc(b):
            return pltpu.make_async_copy(a_hbm.at[idxr.at[b]], bufs.at[b],
                                         gsemA[b])

        def b_desc(b):
            return pltpu.make_async_copy(b_hbm.at[idxc.at[b]], bufs.at[b],
                                         gsemB[b])

        def out_descs(c, b):
            off = outbase + c * _K
            return (pltpu.make_async_copy(bufs.at[b], s_hbm.at[pl.ds(off, _K)],
                                          osem[b]),
                    pltpu.make_async_copy(radbuf.at[b],
                                          rad_hbm.at[pl.ds(off, _K)],
                                          osem[b]))


        def radial(b):
            for j in range(_K // 16):
                ir = idxr[b, pl.ds(j * 16, 16)]
                ic = idxc[b, pl.ds(j * 16, 16)]
                dx = plsc.load_gather(cxv, [ir]) - plsc.load_gather(cxv, [ic])
                dy = plsc.load_gather(cyv, [ir]) - plsc.load_gather(cyv, [ic])
                dz = plsc.load_gather(czv, [ir]) - plsc.load_gather(czv, [ic])
                radbuf[b, pl.ds(j * 16, 16)] = dx * dx + dy * dy + dz * dz

        pltpu.sync_copy(cx_hbm, cxv)
        pltpu.sync_copy(cy_hbm, cyv)
        pltpu.sync_copy(cz_hbm, czv)

        for d in idx_descs(0, 0):
            d.start()
            d.wait()
        for d in idx_descs(1, 1):
            d.start()
        a_desc(0).start()
        for d in idx_descs(1, 1):
            d.wait()
        a_desc(1).start()
        for d in idx_descs(2, 2):
            d.start()
        a_desc(0).wait()
        b_desc(0).start(add=True)

        def step(c, b):
            b_desc(b).wait()
            radial(b)
            for d in out_descs(c, b):
                d.start()

            @pl.when(c + 3 < nch)
            def _():
                for d in idx_descs(c + 3, (b + 3) % _NBUF):
                    d.start()

            b1 = (b + 1) % _NBUF
            b2 = (b + 2) % _NBUF

            @pl.when((c + 2 < nch) & (c >= 2))
            def _():
                for d in out_descs(c - 2, b2):
                    d.wait()

            @pl.when(c + 2 < nch)
            def _():
                for d in idx_descs(c + 2, b2):
                    d.wait()
                a_desc(b2).start()

            @pl.when(c + 1 < nch)
            def _():
                a_desc(b1).wait()
                b_desc(b1).start(add=True)

        def group(g, carry):
            for b in range(_NBUF):
                step(g * _NBUF + b, b)
            return carry

        lax.fori_loop(0, ngrp, group, 0)
        for r in range(nrem):
            c = ngrp * _NBUF + r
            step(c, c % _NBUF)
        for c in range(nch - _NBUF, nch):
            b = c % _NBUF
            for d in out_descs(c, b):
                d.wait()

    return _sc_gather


def _edge_body(s_ref, rad_ref, w2_ref, b1_ref, b2_ref, wr_ref,
               f_ref):
    radial = rad_ref[...]
    u = s_ref[...] + radial * wr_ref[...] + b1_ref[...]
    u = u * jax.nn.sigmoid(u)
    v = jnp.dot(u, w2_ref[...], preferred_element_type=jnp.float32) + b2_ref[...]
    f_ref[...] = v * jax.nn.sigmoid(v)


@functools.cache
def _edge_call(ne):
    return pl.pallas_call(
        _edge_body,
        grid=(ne // _BE,),
        in_specs=[
            pl.BlockSpec((_BE, _D), lambda i: (i, 0)),
            pl.BlockSpec((_BE, 1), lambda i: (i, 0)),
            pl.BlockSpec((_D, _D), lambda i: (0, 0)),
            pl.BlockSpec((1, _D), lambda i: (0, 0)),
            pl.BlockSpec((1, _D), lambda i: (0, 0)),
            pl.BlockSpec((1, _D), lambda i: (0, 0)),
        ],
        out_specs=pl.BlockSpec((_BE, _D), lambda i: (i, 0)),
        out_shape=jax.ShapeDtypeStruct((ne, _D), jnp.float32),
    )


@functools.cache
def _sc_scatter_call(c0, nch):
    ne = nch * _K * _NW
    ngrp, nrem = nch // _NBUF, nch % _NBUF

    @functools.partial(
        pl.kernel,
        out_type=jax.ShapeDtypeStruct((_NC * _N, _D), jnp.float32),
        mesh=_sc_mesh(),
        scratch_types=[
            pltpu.VMEM((_NBUF, _K), jnp.int32),
            pltpu.VMEM((_NBUF, _K, _D), jnp.float32),
            pltpu.VMEM_SHARED((_N, _D), jnp.float32),
        ] + [pltpu.SemaphoreType.DMA] * (2 * _NBUF),
    )
    def _sc_scatter(f_hbm, row_hbm, zero_hbm, agg_hbm, idx, buf, aggsh,
                    *sems):
        lsem = sems[0:_NBUF]
        ssem = sems[_NBUF:2 * _NBUF]
        c = lax.axis_index("c")
        s = lax.axis_index("s")
        wid = s * _NC + c
        inbase = (c0 * _NW + wid * nch) * _K
        fbase = wid * nch * _K

        def load_descs(ch, b):
            return (pltpu.make_async_copy(
                        row_hbm.at[pl.ds(inbase + ch * _K, _K)],
                        idx.at[b], lsem[b]),
                    pltpu.make_async_copy(
                        f_hbm.at[pl.ds(fbase + ch * _K, _K)],
                        buf.at[b], lsem[b]))

        def scat_desc(b):
            return pltpu.make_async_copy(buf.at[b], aggsh.at[idx.at[b]],
                                         ssem[b])

        pltpu.sync_copy(zero_hbm.at[pl.ds(s * _NPT, _NPT)],
                        aggsh.at[pl.ds(s * _NPT, _NPT)])

        @pl.when(s == _NS - 1)
        def _():
            pltpu.sync_copy(zero_hbm.at[pl.ds(_NS * _NPT, _NTAIL)],
                            aggsh.at[pl.ds(_NS * _NPT, _NTAIL)])

        plsc.subcore_barrier()

        for ch in (0, 1):
            for d in load_descs(ch, ch):
                d.start()

        def step(ch, b):
            for d in load_descs(ch, b):
                d.wait()
            scat_desc(b).start(add=True)
            b2 = (b + 2) % _NBUF

            @pl.when((ch + 2 < nch) & (ch >= 2))
            def _():
                scat_desc(b2).wait()

            @pl.when(ch + 2 < nch)
            def _():
                for d in load_descs(ch + 2, b2):
                    d.start()

        def group(g, carry):
            for b in range(_NBUF):
                step(g * _NBUF + b, b)
            return carry

        lax.fori_loop(0, ngrp, group, 0)
        for r in range(nrem):
            ch = ngrp * _NBUF + r
            step(ch, ch % _NBUF)
        for ch in range(nch - _NBUF, nch):
            scat_desc(ch % _NBUF).wait()
        plsc.subcore_barrier()
        pltpu.sync_copy(aggsh.at[pl.ds(s * _NPT, _NPT)],
                        agg_hbm.at[pl.ds(c * _N + s * _NPT, _NPT)])

        @pl.when(s == _NS - 1)
        def _():
            pltpu.sync_copy(aggsh.at[pl.ds(_NS * _NPT, _NTAIL)],
                            agg_hbm.at[pl.ds(c * _N + _NS * _NPT, _NTAIL)])

    return _sc_scatter


_NAGG = 2 * len(_SLICES)


def _node_body(*refs):
    h_ref = refs[0]
    agg_refs = refs[1:1 + _NAGG]
    w1h_ref, w1a_ref, b1_ref, w2_ref, b2_ref, o_ref = refs[1 + _NAGG:]
    hb = h_ref[...]
    agg = agg_refs[0][...]
    for a_ref in agg_refs[1:]:
        agg = agg + a_ref[...]
    u = (jnp.dot(hb, w1h_ref[...], preferred_element_type=jnp.float32)
         + jnp.dot(agg, w1a_ref[...], preferred_element_type=jnp.float32)
         + b1_ref[...])
    u = u * jax.nn.sigmoid(u)
    o_ref[...] = hb + jnp.dot(u, w2_ref[...],
                              preferred_element_type=jnp.float32) + b2_ref[...]


_node_call = pl.pallas_call(
    _node_body,
    grid=(5,),
    in_specs=[pl.BlockSpec((_N // 5, _D), lambda i: (i, 0))] * (1 + _NAGG) + [
        pl.BlockSpec((_D, _D), lambda i: (0, 0)),
        pl.BlockSpec((_D, _D), lambda i: (0, 0)),
        pl.BlockSpec((1, _D), lambda i: (0, 0)),
        pl.BlockSpec((_D, _D), lambda i: (0, 0)),
        pl.BlockSpec((1, _D), lambda i: (0, 0)),
    ],
    out_specs=pl.BlockSpec((_N // 5, _D), lambda i: (i, 0)),
    out_shape=jax.ShapeDtypeStruct((_N, _D), jnp.float32),
)


def kernel(h, edge_index, coord, W_e1, b_e1, W_e2, b_e2, W_n1, b_n1, W_n2,
           b_n2):
    row = edge_index[0]
    col = edge_index[1]
    zeros = jnp.zeros((_N, _D), jnp.float32)
    b1 = b_e1.reshape(1, _D)
    b2 = b_e2.reshape(1, _D)
    wr = W_e1[2 * _D:2 * _D + 1]

    A, B = _pre_call(h, W_e1[0:_D], W_e1[_D:2 * _D])
    aggs = []
    for c0, nch in _SLICES:
        ne = nch * _K * _NW
        ST, rad = _sc_gather_call(c0, nch)(
            A, B, coord[:, 0], coord[:, 1], coord[:, 2], row, col)
        f = _edge_call(ne)(ST, rad.reshape(ne, 1), W_e2, b1, b2, wr)
        agg2 = _sc_scatter_call(c0, nch)(f, row, zeros)
        aggs += [agg2[:_N], agg2[_N:]]
    out = _node_call(h, *aggs, W_n1[:_D], W_n1[_D:], b_n1.reshape(1, _D),
                     W_n2, b_n2.reshape(1, _D))
    return out
